# baseline (speedup 1.0000x reference)
import functools

import jax
import jax.numpy as jnp
from jax import lax
from jax.experimental import pallas as pl
from jax.experimental.pallas import tpu as pltpu
from jax.experimental.pallas import tpu_sc as plsc

B = 4096
L = 200
D = 64
TAO = 0.2
TEMP = 0.5
NPAD = 40
LANES = 16
NC, NS = 2, 16
NW = NC * NS
ROWS_TOTAL = 2 * B
RPW = ROWS_TOTAL // NW
GR = 8
NGRP = RPW // GR
IDXLEN = GR * NPAD
CHUNK = 128
LP = 256

TM = 512
TN = 4096

_mesh = plsc.VectorSubcoreMesh(
    core_axis_name="c", subcore_axis_name="s", num_cores=NC, num_subcores=NS
)


@functools.partial(
    pl.kernel,
    out_type=jax.ShapeDtypeStruct((ROWS_TOTAL, D), jnp.float32),
    mesh=_mesh,
    scratch_types=[
        pltpu.VMEM((RPW, LP), jnp.int32),
        pltpu.VMEM((RPW + LANES,), jnp.int32),
        pltpu.VMEM((RPW + LANES,), jnp.int32),
        pltpu.VMEM((IDXLEN,), jnp.int32),
        pltpu.VMEM((IDXLEN,), jnp.int32),
        pltpu.VMEM((IDXLEN, D), jnp.float32),
        pltpu.VMEM((IDXLEN, D), jnp.float32),
        pltpu.VMEM((RPW, D), jnp.float32),
        pltpu.SemaphoreType.DMA,
        pltpu.SemaphoreType.DMA,
    ],
    compiler_params=pltpu.CompilerParams(
        needs_layout_passes=False, use_tc_tiling_on_sc=False
    ),
)
def _pooled_sc(seq_hbm, start_hbm, sub_hbm, table_hbm, z_hbm,
               seqbuf, start_v, sub_v, idx0, idx1, row0, row1, zbuf,
               sem0, sem1):
    wid = lax.axis_index("s") * NC + lax.axis_index("c")
    base = wid * RPW
    b_base = lax.rem(base, B)

    pltpu.sync_copy(seq_hbm.at[pl.ds(b_base, RPW)], seqbuf)
    pltpu.sync_copy(start_hbm.at[pl.ds(base, RPW)], start_v.at[pl.ds(0, RPW)])
    pltpu.sync_copy(sub_hbm.at[pl.ds(base, RPW)], sub_v.at[pl.ds(0, RPW)])

    lane = lax.iota(jnp.int32, LANES)
    rr_lane = lane % GR
    th_lane = lane // GR

    bufs = ((idx0, row0, sem0), (idx1, row1, sem1))

    def fire(g, par):
        idxbuf, rowbuf, sem = bufs[par]
        r0 = g * GR
        rvec = r0 + rr_lane
        st8 = plsc.load_gather(start_v, [rvec])
        sl8 = plsc.load_gather(sub_v, [rvec])
        for tb in range(0, NPAD, 2):
            tv = tb + th_lane
            pos = jnp.where(tv < sl8, st8 + tv, st8)
            ids = plsc.load_gather(seqbuf, [rvec, pos])
            idxbuf[pl.ds(tb * GR, LANES)] = ids
        for off in range(0, IDXLEN, CHUNK):
            n = min(CHUNK, IDXLEN - off)
            pltpu.async_copy(table_hbm.at[idxbuf.at[pl.ds(off, n)]],
                             rowbuf.at[pl.ds(off, n)], sem)

    def drain(par):
        idxbuf, rowbuf, sem = bufs[par]
        for off in range(0, IDXLEN, CHUNK):
            n = min(CHUNK, IDXLEN - off)
            pltpu.make_async_copy(table_hbm.at[idxbuf.at[pl.ds(off, n)]],
                                  rowbuf.at[pl.ds(off, n)], sem).wait()

    def accumulate(g, par):
        rowbuf = bufs[par][1]
        r0 = g * GR

        def row_body(rr, c2):
            r = r0 + rr
            slv = sub_v[pl.ds(r, LANES)].astype(jnp.float32)
            slf = slv[0]
            cor = jnp.float32(NPAD) - slf
            for p in range(D // LANES):
                cols = pl.ds(LANES * p, LANES)
                first = rowbuf[rr, cols]
                a0 = first
                a1 = rowbuf[GR + rr, cols]
                a2 = rowbuf[2 * GR + rr, cols]
                a3 = rowbuf[3 * GR + rr, cols]
                for t in range(4, NPAD, 4):
                    a0 = a0 + rowbuf[t * GR + rr, cols]
                    a1 = a1 + rowbuf[(t + 1) * GR + rr, cols]
                    a2 = a2 + rowbuf[(t + 2) * GR + rr, cols]
                    a3 = a3 + rowbuf[(t + 3) * GR + rr, cols]
                acc = (a0 + a1) + (a2 + a3)
                zbuf[r, cols] = (acc - cor * first) / slf
            return c2

        lax.fori_loop(0, GR, row_body, 0)

    fire(0, 0)

    def pair_body(k, carry):
        g0 = 2 * k
        fire(g0 + 1, 1)
        drain(0)
        accumulate(g0, 0)

        @pl.when(g0 + 2 < NGRP)
        def _():
            fire(g0 + 2, 0)

        drain(1)
        accumulate(g0 + 1, 1)
        return carry

    lax.fori_loop(0, NGRP // 2, pair_body, 0)
    pltpu.sync_copy(zbuf, z_hbm.at[pl.ds(base, RPW)])


def _sim_body(z_ref, o_ref):
    i = pl.program_id(0)
    j = pl.program_id(1)
    zi = z_ref[pl.ds(i * TM, TM), :]
    zj = z_ref[pl.ds(j * TN, TN), :]
    s = lax.dot_general(zi, zj, (((1,), (1,)), ((), ())),
                        preferred_element_type=jnp.float32)
    row = lax.broadcasted_iota(jnp.int32, (TM, TN), 0) + i * TM
    col = lax.broadcasted_iota(jnp.int32, (TM, TN), 1) + j * TN
    o_ref[...] = jnp.where(row == col, jnp.float32(-1e9), s * jnp.float32(1.0 / TEMP))


def _sim_tc(z):
    return pl.pallas_call(
        _sim_body,
        grid=(ROWS_TOTAL // TM, ROWS_TOTAL // TN),
        in_specs=[pl.BlockSpec((ROWS_TOTAL, D), lambda i, j: (0, 0))],
        out_specs=pl.BlockSpec((TM, TN), lambda i, j: (i, j)),
        out_shape=jax.ShapeDtypeStruct((ROWS_TOTAL, ROWS_TOTAL), jnp.float32),
    )(z)


def kernel(sequences, seqlen, emb_table):
    seqlen = seqlen.astype(jnp.int32)
    u_i = jax.random.uniform(jax.random.key(1), (B,))
    u_j = jax.random.uniform(jax.random.key(2), (B,))
    sub_len = jnp.maximum(1, (TAO * seqlen.astype(jnp.float32)).astype(jnp.int32))
    high = (seqlen - sub_len + 1).astype(jnp.float32)
    start_i = jnp.minimum((u_i * high).astype(jnp.int32), seqlen - sub_len)
    start_j = jnp.minimum((u_j * high).astype(jnp.int32), seqlen - sub_len)
    start_all = jnp.concatenate([start_i, start_j])
    sub_all = jnp.concatenate([sub_len, sub_len])
    seq_pad = jnp.pad(sequences.astype(jnp.int32), ((0, 0), (0, LP - L)))
    z = _pooled_sc(seq_pad, start_all, sub_all, emb_table)
    logits = _sim_tc(z)
    labels = (jnp.arange(ROWS_TOTAL, dtype=jnp.int32) + B) % ROWS_TOTAL
    return logits, labels


# speedup vs baseline: 1.0080x; 1.0080x over previous
import functools

import jax
import jax.numpy as jnp
from jax import lax
from jax.experimental import pallas as pl
from jax.experimental.pallas import tpu as pltpu
from jax.experimental.pallas import tpu_sc as plsc

B = 4096
L = 200
D = 64
TAO = 0.2
TEMP = 0.5
NPAD = 40
LANES = 16
NC, NS = 2, 16
NW = NC * NS
ROWS_TOTAL = 2 * B
RPW = ROWS_TOTAL // NW
GR = 8
NGRP = RPW // GR
IDXLEN = GR * NPAD
CHUNK = 128
LP = 256

TM = 512
TN = 4096

_mesh = plsc.VectorSubcoreMesh(
    core_axis_name="c", subcore_axis_name="s", num_cores=NC, num_subcores=NS
)


@functools.partial(
    pl.kernel,
    out_type=jax.ShapeDtypeStruct((ROWS_TOTAL, D), jnp.float32),
    mesh=_mesh,
    scratch_types=[
        pltpu.VMEM((RPW, L), jnp.int32),
        pltpu.VMEM((RPW,), jnp.int32),
        pltpu.VMEM((RPW,), jnp.float32),
        pltpu.VMEM((RPW + LANES,), jnp.int32),
        pltpu.VMEM((RPW + LANES,), jnp.int32),
        pltpu.VMEM((IDXLEN,), jnp.int32),
        pltpu.VMEM((IDXLEN,), jnp.int32),
        pltpu.VMEM((IDXLEN, D), jnp.float32),
        pltpu.VMEM((IDXLEN, D), jnp.float32),
        pltpu.VMEM((RPW, D), jnp.float32),
        pltpu.SemaphoreType.DMA,
        pltpu.SemaphoreType.DMA,
    ],
    compiler_params=pltpu.CompilerParams(
        needs_layout_passes=False, use_tc_tiling_on_sc=False
    ),
)
def _pooled_sc(seq_hbm, slen_hbm, u_hbm, table_hbm, z_hbm,
               seqbuf, slen_v, u_v, start_v, sub_v, idx0, idx1, row0, row1,
               zbuf, sem0, sem1):
    wid = lax.axis_index("s") * NC + lax.axis_index("c")
    base = wid * RPW
    b_base = lax.rem(base, B)

    pltpu.sync_copy(seq_hbm.at[pl.ds(b_base, RPW)], seqbuf)
    pltpu.sync_copy(slen_hbm.at[pl.ds(b_base, RPW)], slen_v)
    pltpu.sync_copy(u_hbm.at[pl.ds(base, RPW)], u_v)

    for c in range(RPW // LANES):
        seg = pl.ds(c * LANES, LANES)
        sl_i = slen_v[seg]
        sub = jnp.maximum(1, (jnp.float32(TAO) * sl_i.astype(jnp.float32))
                          .astype(jnp.int32))
        high = (sl_i - sub + 1).astype(jnp.float32)
        st = jnp.minimum((u_v[seg] * high).astype(jnp.int32), sl_i - sub)
        start_v[seg] = st
        sub_v[seg] = sub

    lane = lax.iota(jnp.int32, LANES)
    rr_lane = lane % GR
    th_lane = lane // GR

    bufs = ((idx0, row0, sem0), (idx1, row1, sem1))

    def fire(g, par):
        idxbuf, rowbuf, sem = bufs[par]
        r0 = g * GR
        rvec = r0 + rr_lane
        st8 = plsc.load_gather(start_v, [rvec])
        sl8 = plsc.load_gather(sub_v, [rvec])
        for tb in range(0, NPAD, 2):
            tv = tb + th_lane
            pos = jnp.where(tv < sl8, st8 + tv, st8)
            ids = plsc.load_gather(seqbuf, [rvec, pos])
            idxbuf[pl.ds(tb * GR, LANES)] = ids
        for off in range(0, IDXLEN, CHUNK):
            n = min(CHUNK, IDXLEN - off)
            pltpu.async_copy(table_hbm.at[idxbuf.at[pl.ds(off, n)]],
                             rowbuf.at[pl.ds(off, n)], sem)

    def drain(par):
        idxbuf, rowbuf, sem = bufs[par]
        for off in range(0, IDXLEN, CHUNK):
            n = min(CHUNK, IDXLEN - off)
            pltpu.make_async_copy(table_hbm.at[idxbuf.at[pl.ds(off, n)]],
                                  rowbuf.at[pl.ds(off, n)], sem).wait()

    def accumulate(g, par):
        rowbuf = bufs[par][1]
        r0 = g * GR

        def row_body(rr, c2):
            r = r0 + rr
            slv = sub_v[pl.ds(r, LANES)].astype(jnp.float32)
            slf = slv[0]
            cor = jnp.float32(NPAD) - slf
            for p in range(D // LANES):
                cols = pl.ds(LANES * p, LANES)
                first = rowbuf[rr, cols]
                a0 = first
                a1 = rowbuf[GR + rr, cols]
                a2 = rowbuf[2 * GR + rr, cols]
                a3 = rowbuf[3 * GR + rr, cols]
                for t in range(4, NPAD, 4):
                    a0 = a0 + rowbuf[t * GR + rr, cols]
                    a1 = a1 + rowbuf[(t + 1) * GR + rr, cols]
                    a2 = a2 + rowbuf[(t + 2) * GR + rr, cols]
                    a3 = a3 + rowbuf[(t + 3) * GR + rr, cols]
                acc = (a0 + a1) + (a2 + a3)
                zbuf[r, cols] = (acc - cor * first) / slf
            return c2

        lax.fori_loop(0, GR, row_body, 0)

    fire(0, 0)

    def pair_body(k, carry):
        g0 = 2 * k
        fire(g0 + 1, 1)
        drain(0)
        accumulate(g0, 0)

        @pl.when(g0 + 2 < NGRP)
        def _():
            fire(g0 + 2, 0)

        drain(1)
        accumulate(g0 + 1, 1)
        return carry

    lax.fori_loop(0, NGRP // 2, pair_body, 0)
    pltpu.sync_copy(zbuf, z_hbm.at[pl.ds(base, RPW)])


def _sim_body(z_ref, o_ref):
    i = pl.program_id(0)
    j = pl.program_id(1)
    zi = z_ref[pl.ds(i * TM, TM), :]
    zj = z_ref[pl.ds(j * TN, TN), :]
    s = lax.dot_general(zi, zj, (((1,), (1,)), ((), ())),
                        preferred_element_type=jnp.float32)
    row = lax.broadcasted_iota(jnp.int32, (TM, TN), 0) + i * TM
    col = lax.broadcasted_iota(jnp.int32, (TM, TN), 1) + j * TN
    o_ref[...] = jnp.where(row == col, jnp.float32(-1e9), s * jnp.float32(1.0 / TEMP))


def _sim_tc(z):
    return pl.pallas_call(
        _sim_body,
        grid=(ROWS_TOTAL // TM, ROWS_TOTAL // TN),
        in_specs=[pl.BlockSpec((ROWS_TOTAL, D), lambda i, j: (0, 0))],
        out_specs=pl.BlockSpec((TM, TN), lambda i, j: (i, j)),
        out_shape=jax.ShapeDtypeStruct((ROWS_TOTAL, ROWS_TOTAL), jnp.float32),
    )(z)


def kernel(sequences, seqlen, emb_table):
    seqlen = seqlen.astype(jnp.int32)
    u_i = jax.random.uniform(jax.random.key(1), (B,))
    u_j = jax.random.uniform(jax.random.key(2), (B,))
    u_all = jnp.concatenate([u_i, u_j])
    z = _pooled_sc(sequences.astype(jnp.int32), seqlen, u_all, emb_table)
    logits = _sim_tc(z)
    labels = (jnp.arange(ROWS_TOTAL, dtype=jnp.int32) + B) % ROWS_TOTAL
    return logits, labels
